# baseline (speedup 1.0000x reference)
import jax
import jax.numpy as jnp
from jax import lax
from jax.experimental import pallas as pl
from jax.experimental.pallas import tpu as pltpu
from jax.experimental.pallas import tpu_sc as plsc

L = 16
D = 64
HW = 256
NTOK = HW + 1
EPS = 1e-5
NW = 32
BPW = 128
PSTR = 129


def _sc_body(img, dirr, oT, cT, sT, dirtabT, out,
             oT_v, cT_v, sT_v, dirtabT_v, oct_v, strep, dirb, pidx,
             imgb, outb, osem0, osem1):
    ix = lax.iota(jnp.int32, L)
    wid = lax.axis_index("s") * 2 + lax.axis_index("c")
    base_b = wid * BPW

    def full(x):
        return jnp.full((L,), x, dtype=jnp.int32)

    pltpu.sync_copy(oT, oT_v)
    pltpu.sync_copy(cT, cT_v)
    pltpu.sync_copy(sT, sT_v)
    pltpu.sync_copy(dirtabT, dirtabT_v)
    pltpu.sync_copy(dirr.at[pl.ds(base_b, BPW)], dirb)

    def build_oct(f, carry):
        cv = plsc.load_gather(cT_v, [f * 16 + ix])
        for i0 in range(16):
            osc = plsc.load_gather(oT_v, [full(f * 16 + i0)])
            plsc.store_scatter(oct_v, [f * 256 + i0 * 16 + ix], osc + cv)
        return carry
    lax.fori_loop(0, D, build_oct, 0)

    def build_strep(f, carry):
        sv = plsc.load_gather(sT_v, [f * 16 + ix])
        for r in range(16):
            plsc.store_scatter(strep, [full(f * 256 + r) + ix * 16], sv)
        return carry
    lax.fori_loop(0, D, build_strep, 0)

    def build_pidx(item, carry):
        pltpu.sync_copy(img.at[pl.ds((base_b + item) * (HW * 3), HW * 3)],
                        imgb)

        def grp(g, c2):
            g16 = g * L + ix
            tok3 = g16 * 3
            i0 = plsc.load_gather(imgb, [tok3])
            i1 = plsc.load_gather(imgb, [tok3 + 1])
            i2 = plsc.load_gather(imgb, [tok3 + 2])
            pk = i0 * 16 + i1 + i2 * 256
            plsc.store_scatter(pidx, [g16 * PSTR + item], pk)
            return c2
        lax.fori_loop(0, HW // L, grp, 0)
        return carry
    lax.fori_loop(0, BPW, build_pidx, 0)

    def pos_step(it, carry):
        for k, osem in ((0, osem0), (1, osem1)):
            p = 2 * it + k
            koff = k * (D * BPW)

            @pl.when(jnp.logical_and(it >= 1, p - 2 < NTOK))
            def _wait_out():
                pltpu.make_async_copy(
                    outb.at[pl.ds(koff, D * BPW)],
                    out.at[pl.ds(((p - 2) * NW + wid) * (D * BPW), D * BPW)],
                    osem).wait()

            @pl.when(p < HW)
            def _tokens():
                pks = [plsc.load_gather(pidx, [p * PSTR + bg * L + ix])
                       for bg in range(8)]
                cis = [pk & 255 for pk in pks]
                sbs = [lax.shift_right_logical(pk, 8) * 16 + ix for pk in pks]

                def feat(fi, c2):
                    for u in range(4):
                        fo = fi * 4 + u
                        fo256 = full(fo * 256)
                        for bg in range(8):
                            x = (plsc.load_gather(oct_v, [fo256 + cis[bg]])
                                 + plsc.load_gather(strep, [fo256 + sbs[bg]]))
                            outb[pl.ds(koff + fo * BPW + bg * L, L)] = x
                    return c2
                lax.fori_loop(0, D // 4, feat, 0)

            @pl.when(p == HW)
            def _dir_row():
                dgs = [plsc.load_gather(dirb, [bg * L + ix])
                       for bg in range(8)]

                def featd(fi, c2):
                    for u in range(4):
                        fo = fi * 4 + u
                        fo4 = full(fo * 4)
                        for bg in range(8):
                            x = plsc.load_gather(dirtabT_v, [fo4 + dgs[bg]])
                            outb[pl.ds(koff + fo * BPW + bg * L, L)] = x
                    return c2
                lax.fori_loop(0, D // 4, featd, 0)

            @pl.when(p < NTOK)
            def _flush():
                pltpu.async_copy(
                    outb.at[pl.ds(koff, D * BPW)],
                    out.at[pl.ds((p * NW + wid) * (D * BPW), D * BPW)],
                    osem)
        return carry

    lax.fori_loop(0, (NTOK + 1) // 2 + 1, pos_step, 0)


def _tc_body(x_ref, pos_ref, g_ref, b_ref, o_ref):
    v = x_ref[...].reshape(NW, D, BPW)
    pos = pos_ref[...]
    v = v + pos.reshape(1, D, 1)
    m = jnp.mean(v, axis=1, keepdims=True)
    var = jnp.mean(v * v, axis=1, keepdims=True) - m * m
    y = (v - m) * lax.rsqrt(var + EPS)
    y = y * g_ref[...].reshape(1, D, 1) + b_ref[...].reshape(1, D, 1)
    o_ref[...] = jnp.transpose(y, (1, 0, 2)).reshape(1, D, NW * BPW)


def kernel(image, direction, object_tab, color_tab, state_tab, direction_tab,
           position_tab, ln_gamma, ln_beta):
    b, h, w, _ = image.shape
    f32 = jnp.float32
    img2 = image.astype(jnp.int32).reshape(b * h * w * 3)
    d_i = jnp.clip(direction.astype(jnp.int32), 0, 3)

    mesh = plsc.VectorSubcoreMesh(core_axis_name="c", subcore_axis_name="s",
                                  num_cores=2, num_subcores=16)
    sc = pl.kernel(
        _sc_body,
        out_type=jax.ShapeDtypeStruct((NTOK * NW * D * BPW,), f32),
        mesh=mesh,
        compiler_params=pltpu.CompilerParams(needs_layout_passes=False),
        scratch_types=[
            pltpu.VMEM((D * 16,), f32),
            pltpu.VMEM((D * 16,), f32),
            pltpu.VMEM((D * 16,), f32),
            pltpu.VMEM((D * 4,), f32),
            pltpu.VMEM((D * 256,), f32),
            pltpu.VMEM((D * 256,), f32),
            pltpu.VMEM((BPW,), jnp.int32),
            pltpu.VMEM((HW * PSTR,), jnp.int32),
            pltpu.VMEM((HW * 3,), jnp.int32),
            pltpu.VMEM((2 * D * BPW,), f32),
            pltpu.SemaphoreType.DMA,
            pltpu.SemaphoreType.DMA,
        ],
    )
    tok = sc(img2, d_i,
             object_tab.T.reshape(-1), color_tab.T.reshape(-1),
             state_tab.T.reshape(-1), direction_tab.T.reshape(-1))

    ln = pl.pallas_call(
        _tc_body,
        grid=(NTOK,),
        in_specs=[
            pl.BlockSpec((NW * D * BPW,), lambda i: (i,)),
            pl.BlockSpec((1, 1, D), lambda i: (i, 0, 0)),
            pl.BlockSpec((1, D), lambda i: (0, 0)),
            pl.BlockSpec((1, D), lambda i: (0, 0)),
        ],
        out_specs=pl.BlockSpec((1, D, NW * BPW), lambda i: (i, 0, 0)),
        out_shape=jax.ShapeDtypeStruct((NTOK, D, b), f32),
    )(tok, position_tab.reshape(NTOK, 1, D),
      ln_gamma.reshape(1, D), ln_beta.reshape(1, D))

    return jnp.transpose(ln, (2, 0, 1))


# speedup vs baseline: 3.9437x; 3.9437x over previous
import jax
import jax.numpy as jnp
from jax import lax
from jax.experimental import pallas as pl
from jax.experimental.pallas import tpu as pltpu
from jax.experimental.pallas import tpu_sc as plsc

L = 16
D = 64
HW = 256
NTOK = HW + 1
EPS = 1e-5
NW = 32
BPW = 128
PSTR = 129


def _sc_body(cio, dirr, oT, cT, sT, dirtabT, out,
             oT_v, cT_v, sT_v, dirtabT_v, oct_v, strep, dirb, pidx2,
             outb, osem0, osem1):
    ix = lax.iota(jnp.int32, L)
    wid = lax.axis_index("s") * 2 + lax.axis_index("c")
    base_b = wid * BPW

    def full(x):
        return jnp.full((L,), x, dtype=jnp.int32)

    pltpu.sync_copy(oT, oT_v)
    pltpu.sync_copy(cT, cT_v)
    pltpu.sync_copy(sT, sT_v)
    pltpu.sync_copy(dirtabT, dirtabT_v)
    pltpu.sync_copy(dirr.at[pl.ds(base_b, BPW)], dirb)
    pltpu.sync_copy(cio.at[:, pl.ds(base_b, BPW)], pidx2)

    def build_oct(f, carry):
        cv = plsc.load_gather(cT_v, [f * 16 + ix])
        for i0 in range(16):
            osc = plsc.load_gather(oT_v, [full(f * 16 + i0)])
            plsc.store_scatter(oct_v, [f * 256 + i0 * 16 + ix], osc + cv)
        return carry
    lax.fori_loop(0, D, build_oct, 0)

    def build_strep(f, carry):
        sv = plsc.load_gather(sT_v, [f * 16 + ix])
        for r in range(16):
            plsc.store_scatter(strep, [full(f * 256 + r) + ix * 16], sv)
        return carry
    lax.fori_loop(0, D, build_strep, 0)

    def pos_step(it, carry):
        for k, osem in ((0, osem0), (1, osem1)):
            p = 2 * it + k
            koff = k * (D * BPW)

            @pl.when(jnp.logical_and(it >= 1, p - 2 < NTOK))
            def _wait_out():
                pltpu.make_async_copy(
                    outb.at[pl.ds(koff, D * BPW)],
                    out.at[pl.ds(((p - 2) * NW + wid) * (D * BPW), D * BPW)],
                    osem).wait()

            @pl.when(p < HW)
            def _tokens():
                pks = [plsc.load_gather(pidx2, [full(p), bg * L + ix])
                       for bg in range(8)]
                cis = [pk & 255 for pk in pks]
                sbs = [lax.shift_right_logical(pk, 8) * 16 + ix for pk in pks]

                def feat(fi, c2):
                    for u in range(4):
                        fo = fi * 4 + u
                        fo256 = full(fo * 256)
                        for bg in range(8):
                            x = (plsc.load_gather(oct_v, [fo256 + cis[bg]])
                                 + plsc.load_gather(strep, [fo256 + sbs[bg]]))
                            outb[pl.ds(koff + fo * BPW + bg * L, L)] = x
                    return c2
                lax.fori_loop(0, D // 4, feat, 0)

            @pl.when(p == HW)
            def _dir_row():
                dgs = [plsc.load_gather(dirb, [bg * L + ix])
                       for bg in range(8)]

                def featd(fi, c2):
                    for u in range(4):
                        fo = fi * 4 + u
                        fo4 = full(fo * 4)
                        for bg in range(8):
                            x = plsc.load_gather(dirtabT_v, [fo4 + dgs[bg]])
                            outb[pl.ds(koff + fo * BPW + bg * L, L)] = x
                    return c2
                lax.fori_loop(0, D // 4, featd, 0)

            @pl.when(p < NTOK)
            def _flush():
                pltpu.async_copy(
                    outb.at[pl.ds(koff, D * BPW)],
                    out.at[pl.ds((p * NW + wid) * (D * BPW), D * BPW)],
                    osem)
        return carry

    lax.fori_loop(0, (NTOK + 1) // 2 + 1, pos_step, 0)


def _tc_body(x_ref, pos_ref, g_ref, b_ref, o_ref):
    v = x_ref[...].reshape(NW, D, BPW)
    pos = pos_ref[...]
    v = v + pos.reshape(1, D, 1)
    m = jnp.mean(v, axis=1, keepdims=True)
    var = jnp.mean(v * v, axis=1, keepdims=True) - m * m
    y = (v - m) * lax.rsqrt(var + EPS)
    y = y * g_ref[...].reshape(1, D, 1) + b_ref[...].reshape(1, D, 1)
    o_ref[...] = jnp.transpose(y, (1, 0, 2)).reshape(1, D, NW * BPW)


def kernel(image, direction, object_tab, color_tab, state_tab, direction_tab,
           position_tab, ln_gamma, ln_beta):
    b, h, w, _ = image.shape
    f32 = jnp.float32
    im = image.astype(jnp.int32)
    cio = (im[..., 0] * 16 + im[..., 1] + im[..., 2] * 256)
    cio = cio.reshape(b, h * w).T
    d_i = jnp.clip(direction.astype(jnp.int32), 0, 3)

    mesh = plsc.VectorSubcoreMesh(core_axis_name="c", subcore_axis_name="s",
                                  num_cores=2, num_subcores=16)
    sc = pl.kernel(
        _sc_body,
        out_type=jax.ShapeDtypeStruct((NTOK * NW * D * BPW,), f32),
        mesh=mesh,
        compiler_params=pltpu.CompilerParams(needs_layout_passes=False),
        scratch_types=[
            pltpu.VMEM((D * 16,), f32),
            pltpu.VMEM((D * 16,), f32),
            pltpu.VMEM((D * 16,), f32),
            pltpu.VMEM((D * 4,), f32),
            pltpu.VMEM((D * 256,), f32),
            pltpu.VMEM((D * 256,), f32),
            pltpu.VMEM((BPW,), jnp.int32),
            pltpu.VMEM((HW, BPW), jnp.int32),
            pltpu.VMEM((2 * D * BPW,), f32),
            pltpu.SemaphoreType.DMA,
            pltpu.SemaphoreType.DMA,
        ],
    )
    tok = sc(cio, d_i,
             object_tab.T.reshape(-1), color_tab.T.reshape(-1),
             state_tab.T.reshape(-1), direction_tab.T.reshape(-1))

    ln = pl.pallas_call(
        _tc_body,
        grid=(NTOK,),
        in_specs=[
            pl.BlockSpec((NW * D * BPW,), lambda i: (i,)),
            pl.BlockSpec((1, 1, D), lambda i: (i, 0, 0)),
            pl.BlockSpec((1, D), lambda i: (0, 0)),
            pl.BlockSpec((1, D), lambda i: (0, 0)),
        ],
        out_specs=pl.BlockSpec((1, D, NW * BPW), lambda i: (i, 0, 0)),
        out_shape=jax.ShapeDtypeStruct((NTOK, D, b), f32),
    )(tok, position_tab.reshape(NTOK, 1, D),
      ln_gamma.reshape(1, D), ln_beta.reshape(1, D))

    return jnp.transpose(ln, (2, 0, 1))


# speedup vs baseline: 3.9524x; 1.0022x over previous
import jax
import jax.numpy as jnp
from jax import lax
from jax.experimental import pallas as pl
from jax.experimental.pallas import tpu as pltpu
from jax.experimental.pallas import tpu_sc as plsc

L = 16
D = 64
HW = 256
NTOK = HW + 1
EPS = 1e-5
NW = 32
BPW = 128
PSTR = 129


def _sc_body(cio, dirr, oT, cT, sT, dirtabT, out,
             oT_v, cT_v, sT_v, dirtabT_v, oct_v, strep, dirb, pidx2,
             outb, osem0, osem1):
    ix = lax.iota(jnp.int32, L)
    wid = lax.axis_index("s") * 2 + lax.axis_index("c")
    base_b = wid * BPW

    def full(x):
        return jnp.full((L,), x, dtype=jnp.int32)

    pltpu.sync_copy(oT, oT_v)
    pltpu.sync_copy(cT, cT_v)
    pltpu.sync_copy(sT, sT_v)
    pltpu.sync_copy(dirtabT, dirtabT_v)
    pltpu.sync_copy(dirr.at[pl.ds(base_b, BPW)], dirb)
    pltpu.sync_copy(cio.at[:, pl.ds(base_b, BPW)], pidx2)

    def build_oct(f, carry):
        cv = plsc.load_gather(cT_v, [f * 16 + ix])
        for i0 in range(16):
            osc = plsc.load_gather(oT_v, [full(f * 16 + i0)])
            plsc.store_scatter(oct_v, [f * 256 + i0 * 16 + ix], osc + cv)
        return carry
    lax.fori_loop(0, D, build_oct, 0)

    def build_strep(f, carry):
        sv = plsc.load_gather(sT_v, [f * 16 + ix])
        for r in range(16):
            plsc.store_scatter(strep, [full(f * 256 + r) + ix * 16], sv)
        return carry
    lax.fori_loop(0, D, build_strep, 0)

    def pos_step(it, carry):
        for k, osem in ((0, osem0), (1, osem1)):
            p = 2 * it + k
            koff = k * (D * BPW)

            @pl.when(jnp.logical_and(it >= 1, p - 2 < NTOK))
            def _wait_out():
                pltpu.make_async_copy(
                    outb.at[pl.ds(koff, D * BPW)],
                    out.at[pl.ds(((p - 2) * NW + wid) * (D * BPW), D * BPW)],
                    osem).wait()

            @pl.when(p < HW)
            def _tokens():
                pks = [plsc.load_gather(pidx2, [full(p), bg * L + ix])
                       for bg in range(8)]
                cis = [pk & 255 for pk in pks]
                sbs = [lax.shift_right_logical(pk, 8) * 16 + ix for pk in pks]

                def feat(fi, c2):
                    for u in range(8):
                        fo = fi * 8 + u
                        fo256 = full(fo * 256)
                        for bg in range(8):
                            x = (plsc.load_gather(oct_v, [fo256 + cis[bg]])
                                 + plsc.load_gather(strep, [fo256 + sbs[bg]]))
                            outb[pl.ds(koff + fo * BPW + bg * L, L)] = x
                    return c2
                lax.fori_loop(0, D // 8, feat, 0)

            @pl.when(p == HW)
            def _dir_row():
                dgs = [plsc.load_gather(dirb, [bg * L + ix])
                       for bg in range(8)]

                def featd(fi, c2):
                    for u in range(4):
                        fo = fi * 4 + u
                        fo4 = full(fo * 4)
                        for bg in range(8):
                            x = plsc.load_gather(dirtabT_v, [fo4 + dgs[bg]])
                            outb[pl.ds(koff + fo * BPW + bg * L, L)] = x
                    return c2
                lax.fori_loop(0, D // 4, featd, 0)

            @pl.when(p < NTOK)
            def _flush():
                pltpu.async_copy(
                    outb.at[pl.ds(koff, D * BPW)],
                    out.at[pl.ds((p * NW + wid) * (D * BPW), D * BPW)],
                    osem)
        return carry

    lax.fori_loop(0, (NTOK + 1) // 2 + 1, pos_step, 0)


def _tc_body(x_ref, pos_ref, g_ref, b_ref, o_ref):
    v = x_ref[...].reshape(NW, D, BPW)
    pos = pos_ref[...]
    v = v + pos.reshape(1, D, 1)
    m = jnp.mean(v, axis=1, keepdims=True)
    var = jnp.mean(v * v, axis=1, keepdims=True) - m * m
    y = (v - m) * lax.rsqrt(var + EPS)
    y = y * g_ref[...].reshape(1, D, 1) + b_ref[...].reshape(1, D, 1)
    o_ref[...] = jnp.transpose(y, (1, 0, 2)).reshape(1, D, NW * BPW)


def kernel(image, direction, object_tab, color_tab, state_tab, direction_tab,
           position_tab, ln_gamma, ln_beta):
    b, h, w, _ = image.shape
    f32 = jnp.float32
    im = image.astype(jnp.int32)
    cio = (im[..., 0] * 16 + im[..., 1] + im[..., 2] * 256)
    cio = cio.reshape(b, h * w).T
    d_i = jnp.clip(direction.astype(jnp.int32), 0, 3)

    mesh = plsc.VectorSubcoreMesh(core_axis_name="c", subcore_axis_name="s",
                                  num_cores=2, num_subcores=16)
    sc = pl.kernel(
        _sc_body,
        out_type=jax.ShapeDtypeStruct((NTOK * NW * D * BPW,), f32),
        mesh=mesh,
        compiler_params=pltpu.CompilerParams(needs_layout_passes=False),
        scratch_types=[
            pltpu.VMEM((D * 16,), f32),
            pltpu.VMEM((D * 16,), f32),
            pltpu.VMEM((D * 16,), f32),
            pltpu.VMEM((D * 4,), f32),
            pltpu.VMEM((D * 256,), f32),
            pltpu.VMEM((D * 256,), f32),
            pltpu.VMEM((BPW,), jnp.int32),
            pltpu.VMEM((HW, BPW), jnp.int32),
            pltpu.VMEM((2 * D * BPW,), f32),
            pltpu.SemaphoreType.DMA,
            pltpu.SemaphoreType.DMA,
        ],
    )
    tok = sc(cio, d_i,
             object_tab.T.reshape(-1), color_tab.T.reshape(-1),
             state_tab.T.reshape(-1), direction_tab.T.reshape(-1))

    ln = pl.pallas_call(
        _tc_body,
        grid=(NTOK,),
        in_specs=[
            pl.BlockSpec((NW * D * BPW,), lambda i: (i,)),
            pl.BlockSpec((1, 1, D), lambda i: (i, 0, 0)),
            pl.BlockSpec((1, D), lambda i: (0, 0)),
            pl.BlockSpec((1, D), lambda i: (0, 0)),
        ],
        out_specs=pl.BlockSpec((1, D, NW * BPW), lambda i: (i, 0, 0)),
        out_shape=jax.ShapeDtypeStruct((NTOK, D, b), f32),
    )(tok, position_tab.reshape(NTOK, 1, D),
      ln_gamma.reshape(1, D), ln_beta.reshape(1, D))

    return jnp.transpose(ln, (2, 0, 1))
